# Initial kernel scaffold; baseline (speedup 1.0000x reference)
import jax
import jax.numpy as jnp
from jax.experimental import pallas as pl


def kernel(x, edge, ln_g, ln_b, W1, b1, W2, b2, W3, b3):
    raise NotImplementedError("write your pallas kernel here")


# speedup vs baseline: 9.4142x; 9.4142x over previous
import functools

import jax
import jax.numpy as jnp
from jax import lax
from jax.experimental import pallas as pl
from jax.experimental.pallas import tpu as pltpu
from jax.experimental.pallas import tpu_sc as plsc

N = 10000
NPAD = 10240
E = 320000
NC = 2
NS = 16
NW = NC * NS
EPW = E // NW
K = 80
ITERS = EPW // K
RPT = NPAD // NS
ZROWS = 128
B = 2000
EB = 2000
NEB = E // EB
EPS = 1e-5


@functools.lru_cache(maxsize=None)
def _make_agg(D):
    mesh = plsc.VectorSubcoreMesh(
        core_axis_name="c", subcore_axis_name="s",
        num_cores=NC, num_subcores=NS)

    @functools.partial(
        pl.kernel,
        out_type=jax.ShapeDtypeStruct((NC, NPAD, D), jnp.float32),
        mesh=mesh,
        compiler_params=pltpu.CompilerParams(use_tc_tiling_on_sc=False),
        scratch_types=[
            pltpu.VMEM((K,), jnp.int32),
            pltpu.VMEM((K,), jnp.int32),
            pltpu.VMEM((K, D), jnp.float32),
            pltpu.VMEM((ZROWS, D), jnp.float32),
            pltpu.VMEM_SHARED((NPAD, D), jnp.float32),
            pltpu.SemaphoreType.DMA,
        ],
    )
    def agg(table_hbm, src_hbm, dst_hbm, out_hbm,
            src_v, dst_v, rows_v, zb_v, acc, sem):
        c = lax.axis_index("c")
        s = lax.axis_index("s")
        wid = c * NS + s

        zeros16 = jnp.zeros((16,), jnp.float32)

        def zrow(j, _):
            def zlane(k, _):
                zb_v[j, pl.ds(k * 16, 16)] = zeros16
                return None
            return lax.fori_loop(0, D // 16, zlane, None)
        lax.fori_loop(0, ZROWS, zrow, None)

        def zcopy(j, _):
            pltpu.sync_copy(zb_v, acc.at[pl.ds(s * RPT + j * ZROWS, ZROWS)])
            return None
        lax.fori_loop(0, RPT // ZROWS, zcopy, None)
        plsc.subcore_barrier()

        base = wid * EPW

        def step(i, _):
            off = pl.multiple_of(base + i * K, 8)
            pltpu.sync_copy(src_hbm.at[pl.ds(off, K)], src_v)
            pltpu.sync_copy(dst_hbm.at[pl.ds(off, K)], dst_v)
            pltpu.async_copy(table_hbm.at[src_v], rows_v, sem).wait()
            pltpu.sync_copy(rows_v, acc.at[dst_v], add=True)
            return None
        lax.fori_loop(0, ITERS, step, None)
        plsc.subcore_barrier()

        pltpu.sync_copy(acc.at[pl.ds(s * RPT, RPT)],
                        out_hbm.at[c, pl.ds(s * RPT, RPT)])

    return agg


def _deg_body(dst_ref, dinv_ref):
    i = pl.program_id(0)
    d2 = dst_ref[...]
    qi = lax.broadcasted_iota(jnp.int32, (EB, 128), 1)
    hi = (lax.shift_right_logical(d2, 7) == qi).astype(jnp.bfloat16)
    lo = ((d2 & 127) == qi).astype(jnp.bfloat16)
    cnt = lax.dot_general(hi, lo, (((0,), (0,)), ((), ())),
                          preferred_element_type=jnp.float32)

    @pl.when(i == 0)
    def _():
        dinv_ref[...] = jnp.zeros_like(dinv_ref)

    dinv_ref[...] += cnt

    @pl.when(i == NEB - 1)
    def _():
        dinv_ref[...] = lax.rsqrt(dinv_ref[...] + 1.0)


def _deg_dinv(dst_col):
    return pl.pallas_call(
        _deg_body,
        grid=(NEB,),
        in_specs=[pl.BlockSpec((EB, 1), lambda i: (i, 0))],
        out_specs=pl.BlockSpec((128, 128), lambda i: (0, 0)),
        out_shape=jax.ShapeDtypeStruct((128, 128), jnp.float32),
    )(dst_col)


def _tc1_body(x_ref, g_ref, b_ref, dv_ref, w_ref, hp_ref, s_ref):
    xb = x_ref[...]
    mu = jnp.mean(xb, axis=1, keepdims=True)
    xc = xb - mu
    var = jnp.mean(xc * xc, axis=1, keepdims=True)
    xn = xc * lax.rsqrt(var + EPS) * g_ref[...] + b_ref[...]
    dinv = dv_ref[...]
    hp = jnp.dot(xn * dinv, w_ref[...], preferred_element_type=jnp.float32)
    hp_ref[...] = hp
    s_ref[...] = hp * dinv


def _mid_body(ra_ref, rb_ref, sin_ref, bias_ref, dv_ref, w_ref,
              hp_ref, s_ref):
    dinv = dv_ref[...]
    u = jnp.maximum(
        dinv * (ra_ref[...] + rb_ref[...]) + sin_ref[...] + bias_ref[...],
        0.0)
    hp = jnp.dot(u * dinv, w_ref[...], preferred_element_type=jnp.float32)
    hp_ref[...] = hp
    s_ref[...] = hp * dinv


def _fin_body(ra_ref, rb_ref, sin_ref, bias_ref, dv_ref, out_ref):
    dinv = dv_ref[...]
    out_ref[...] = (dinv * (ra_ref[...] + rb_ref[...])
                    + sin_ref[...] + bias_ref[...])


def _row_spec(d):
    return pl.BlockSpec((B, d), lambda i: (i, 0))


def _full_spec(shape):
    return pl.BlockSpec(shape, lambda i: (0,) * len(shape))


def _tc1(x, g2, b2, dinv, W1):
    return pl.pallas_call(
        _tc1_body,
        grid=(N // B,),
        in_specs=[_row_spec(128), _full_spec((1, 128)), _full_spec((1, 128)),
                  _row_spec(1), _full_spec((128, 128))],
        out_specs=[_row_spec(128), _row_spec(128)],
        out_shape=[jax.ShapeDtypeStruct((N, 128), jnp.float32),
                   jax.ShapeDtypeStruct((N, 128), jnp.float32)],
    )(x, g2, b2, dinv, W1)


def _tc_mid(ra, rb, sin, bias2, dinv, W, dout):
    return pl.pallas_call(
        _mid_body,
        grid=(N // B,),
        in_specs=[_row_spec(128), _row_spec(128), _row_spec(128),
                  _full_spec((1, 128)), _row_spec(1),
                  _full_spec((128, dout))],
        out_specs=[_row_spec(dout), _row_spec(dout)],
        out_shape=[jax.ShapeDtypeStruct((N, dout), jnp.float32),
                   jax.ShapeDtypeStruct((N, dout), jnp.float32)],
    )(ra, rb, sin, bias2, dinv, W)


def _tc_fin(ra, rb, sin, bias2, dinv):
    return pl.pallas_call(
        _fin_body,
        grid=(N // B,),
        in_specs=[_row_spec(32), _row_spec(32), _row_spec(32),
                  _full_spec((1, 32)), _row_spec(1)],
        out_specs=pl.BlockSpec((B, 32), lambda i: (i, 0)),
        out_shape=jax.ShapeDtypeStruct((N, 32), jnp.float32),
    )(ra, rb, sin, bias2, dinv)


def kernel(x, edge, ln_g, ln_b, W1, b1, W2, b2, W3, b3):
    src = edge[0]
    dst = edge[1]
    g2 = ln_g.reshape(1, 128)
    lb2 = ln_b.reshape(1, 128)
    b1_2 = b1.reshape(1, 128)
    b2_2 = b2.reshape(1, 128)
    b3_2 = b3.reshape(1, 32)

    dinv2d = _deg_dinv(dst.reshape(E, 1))
    dinv = dinv2d.reshape(128 * 128, 1)[:N]

    h1p, s1 = _tc1(x, g2, lb2, dinv, W1)
    r1 = _make_agg(128)(h1p, src, dst)
    h2p, s2 = _tc_mid(r1[0], r1[1], s1, b1_2, dinv, W2, 128)
    r2 = _make_agg(128)(h2p, src, dst)
    h3p, s3 = _tc_mid(r2[0], r2[1], s2, b2_2, dinv, W3, 32)
    r3 = _make_agg(32)(h3p, src, dst)
    return _tc_fin(r3[0], r3[1], s3, b3_2, dinv)


# speedup vs baseline: 15.2450x; 1.6194x over previous
import functools

import jax
import jax.numpy as jnp
from jax import lax
from jax.experimental import pallas as pl
from jax.experimental.pallas import tpu as pltpu
from jax.experimental.pallas import tpu_sc as plsc

N = 10000
NPAD = 10240
E = 320000
NC = 2
NS = 16
NW = NC * NS
EPW = E // NW
K = 80
ITERS = EPW // K
RPT = NPAD // NS
ZROWS = 128
B = 2000
EB = 2000
NEB = E // EB
EPS = 1e-5


@functools.lru_cache(maxsize=None)
def _make_agg(D):
    mesh = plsc.VectorSubcoreMesh(
        core_axis_name="c", subcore_axis_name="s",
        num_cores=NC, num_subcores=NS)

    @functools.partial(
        pl.kernel,
        out_type=jax.ShapeDtypeStruct((NC, NPAD, D), jnp.float32),
        mesh=mesh,
        compiler_params=pltpu.CompilerParams(use_tc_tiling_on_sc=False),
        scratch_types=[
            pltpu.VMEM((2, K), jnp.int32),
            pltpu.VMEM((2, K), jnp.int32),
            pltpu.VMEM((K, D), jnp.float32),
            pltpu.VMEM((K, D), jnp.float32),
            pltpu.VMEM((ZROWS, D), jnp.float32),
            pltpu.VMEM_SHARED((NPAD, D), jnp.float32),
            pltpu.SemaphoreType.DMA,
            pltpu.SemaphoreType.DMA,
            pltpu.SemaphoreType.DMA,
            pltpu.SemaphoreType.DMA,
        ],
    )
    def agg(table_hbm, src_hbm, dst_hbm, out_hbm,
            src_v, dst_v, rows_a, rows_b, zb_v, acc,
            sia, sib, sga, sgb):
        c = lax.axis_index("c")
        s = lax.axis_index("s")
        wid = c * NS + s

        zeros16 = jnp.zeros((16,), jnp.float32)

        def zrow(j, _):
            def zlane(k, _):
                zb_v[j, pl.ds(k * 16, 16)] = zeros16
                return None
            return lax.fori_loop(0, D // 16, zlane, None)
        lax.fori_loop(0, ZROWS, zrow, None)

        def zcopy(j, _):
            pltpu.sync_copy(zb_v, acc.at[pl.ds(s * RPT + j * ZROWS, ZROWS)])
            return None
        lax.fori_loop(0, RPT // ZROWS, zcopy, None)
        plsc.subcore_barrier()

        base = wid * EPW
        last = base + (ITERS - 1) * K

        def issue_idx(i, buf, sem):
            off = pl.multiple_of(
                jnp.minimum(base + i * K, last).astype(jnp.int32), 8)
            pltpu.async_copy(src_hbm.at[pl.ds(off, K)], src_v.at[buf], sem)
            pltpu.async_copy(dst_hbm.at[pl.ds(off, K)], dst_v.at[buf], sem)

        def wait_idx(buf, sem):
            pltpu.make_async_copy(src_hbm.at[pl.ds(0, K)], src_v.at[buf],
                                  sem).wait()
            pltpu.make_async_copy(dst_hbm.at[pl.ds(0, K)], dst_v.at[buf],
                                  sem).wait()

        issue_idx(0, 0, sia)
        issue_idx(1, 1, sib)
        wait_idx(0, sia)
        pltpu.async_copy(table_hbm.at[src_v.at[0]], rows_a, sga)

        def step(j, _):
            i0 = 2 * j
            wait_idx(1, sib)
            pltpu.async_copy(table_hbm.at[src_v.at[1]], rows_b, sgb)
            pltpu.make_async_copy(table_hbm.at[src_v.at[0]], rows_a,
                                  sga).wait()
            pltpu.sync_copy(rows_a, acc.at[dst_v.at[0]], add=True)
            issue_idx(i0 + 2, 0, sia)
            wait_idx(0, sia)
            pltpu.async_copy(table_hbm.at[src_v.at[0]], rows_a, sga)
            pltpu.make_async_copy(table_hbm.at[src_v.at[1]], rows_b,
                                  sgb).wait()
            pltpu.sync_copy(rows_b, acc.at[dst_v.at[1]], add=True)
            issue_idx(i0 + 3, 1, sib)
            return None
        lax.fori_loop(0, (ITERS - 1) // 2, step, None)

        pltpu.make_async_copy(table_hbm.at[src_v.at[0]], rows_a, sga).wait()
        pltpu.sync_copy(rows_a, acc.at[dst_v.at[0]], add=True)
        wait_idx(1, sib)
        plsc.subcore_barrier()

        pltpu.sync_copy(acc.at[pl.ds(s * RPT, RPT)],
                        out_hbm.at[c, pl.ds(s * RPT, RPT)])

    return agg


def _deg_body(dst_ref, dinv_ref):
    i = pl.program_id(0)
    d2 = dst_ref[...]
    qi = lax.broadcasted_iota(jnp.int32, (EB, 128), 1)
    hi = (lax.shift_right_logical(d2, 7) == qi).astype(jnp.bfloat16)
    lo = ((d2 & 127) == qi).astype(jnp.bfloat16)
    cnt = lax.dot_general(hi, lo, (((0,), (0,)), ((), ())),
                          preferred_element_type=jnp.float32)

    @pl.when(i == 0)
    def _():
        dinv_ref[...] = jnp.zeros_like(dinv_ref)

    dinv_ref[...] += cnt

    @pl.when(i == NEB - 1)
    def _():
        dinv_ref[...] = lax.rsqrt(dinv_ref[...] + 1.0)


def _deg_dinv(dst_col):
    return pl.pallas_call(
        _deg_body,
        grid=(NEB,),
        in_specs=[pl.BlockSpec((EB, 1), lambda i: (i, 0))],
        out_specs=pl.BlockSpec((128, 128), lambda i: (0, 0)),
        out_shape=jax.ShapeDtypeStruct((128, 128), jnp.float32),
    )(dst_col)


def _tc1_body(x_ref, g_ref, b_ref, dv_ref, w_ref, hp_ref, s_ref):
    xb = x_ref[...]
    mu = jnp.mean(xb, axis=1, keepdims=True)
    xc = xb - mu
    var = jnp.mean(xc * xc, axis=1, keepdims=True)
    xn = xc * lax.rsqrt(var + EPS) * g_ref[...] + b_ref[...]
    dinv = dv_ref[...]
    hp = jnp.dot(xn * dinv, w_ref[...], preferred_element_type=jnp.float32)
    hp_ref[...] = hp
    s_ref[...] = hp * dinv


def _mid_body(ra_ref, rb_ref, sin_ref, bias_ref, dv_ref, w_ref,
              hp_ref, s_ref):
    dinv = dv_ref[...]
    u = jnp.maximum(
        dinv * (ra_ref[...] + rb_ref[...]) + sin_ref[...] + bias_ref[...],
        0.0)
    hp = jnp.dot(u * dinv, w_ref[...], preferred_element_type=jnp.float32)
    hp_ref[...] = hp
    s_ref[...] = hp * dinv


def _fin_body(ra_ref, rb_ref, sin_ref, bias_ref, dv_ref, out_ref):
    dinv = dv_ref[...]
    out_ref[...] = (dinv * (ra_ref[...] + rb_ref[...])
                    + sin_ref[...] + bias_ref[...])


def _row_spec(d):
    return pl.BlockSpec((B, d), lambda i: (i, 0))


def _full_spec(shape):
    return pl.BlockSpec(shape, lambda i: (0,) * len(shape))


def _tc1(x, g2, b2, dinv, W1):
    return pl.pallas_call(
        _tc1_body,
        grid=(N // B,),
        in_specs=[_row_spec(128), _full_spec((1, 128)), _full_spec((1, 128)),
                  _row_spec(1), _full_spec((128, 128))],
        out_specs=[_row_spec(128), _row_spec(128)],
        out_shape=[jax.ShapeDtypeStruct((N, 128), jnp.float32),
                   jax.ShapeDtypeStruct((N, 128), jnp.float32)],
    )(x, g2, b2, dinv, W1)


def _tc_mid(ra, rb, sin, bias2, dinv, W, dout):
    return pl.pallas_call(
        _mid_body,
        grid=(N // B,),
        in_specs=[_row_spec(128), _row_spec(128), _row_spec(128),
                  _full_spec((1, 128)), _row_spec(1),
                  _full_spec((128, dout))],
        out_specs=[_row_spec(dout), _row_spec(dout)],
        out_shape=[jax.ShapeDtypeStruct((N, dout), jnp.float32),
                   jax.ShapeDtypeStruct((N, dout), jnp.float32)],
    )(ra, rb, sin, bias2, dinv, W)


def _tc_fin(ra, rb, sin, bias2, dinv):
    return pl.pallas_call(
        _fin_body,
        grid=(N // B,),
        in_specs=[_row_spec(32), _row_spec(32), _row_spec(32),
                  _full_spec((1, 32)), _row_spec(1)],
        out_specs=pl.BlockSpec((B, 32), lambda i: (i, 0)),
        out_shape=jax.ShapeDtypeStruct((N, 32), jnp.float32),
    )(ra, rb, sin, bias2, dinv)


def kernel(x, edge, ln_g, ln_b, W1, b1, W2, b2, W3, b3):
    src = edge[0]
    dst = edge[1]
    g2 = ln_g.reshape(1, 128)
    lb2 = ln_b.reshape(1, 128)
    b1_2 = b1.reshape(1, 128)
    b2_2 = b2.reshape(1, 128)
    b3_2 = b3.reshape(1, 32)

    dinv2d = _deg_dinv(dst.reshape(E, 1))
    dinv = dinv2d.reshape(128 * 128, 1)[:N]

    h1p, s1 = _tc1(x, g2, lb2, dinv, W1)
    r1 = _make_agg(128)(h1p, src, dst)
    h2p, s2 = _tc_mid(r1[0], r1[1], s1, b1_2, dinv, W2, 128)
    r2 = _make_agg(128)(h2p, src, dst)
    h3p, s3 = _tc_mid(r2[0], r2[1], s2, b2_2, dinv, W3, 32)
    r3 = _make_agg(32)(h3p, src, dst)
    return _tc_fin(r3[0], r3[1], s3, b3_2, dinv)


# speedup vs baseline: 19.8398x; 1.3014x over previous
import functools

import jax
import jax.numpy as jnp
from jax import lax
from jax.experimental import pallas as pl
from jax.experimental.pallas import tpu as pltpu
from jax.experimental.pallas import tpu_sc as plsc

N = 10000
NPAD = 10240
E = 320000
NC = 2
NS = 16
NW = NC * NS
EPW = E // NW
K = 80
ITERS = EPW // K
RPT = NPAD // NS
ZROWS = 128
B = 2000
EB = 2000
NEB = E // EB
EPS = 1e-5


@functools.lru_cache(maxsize=None)
def _make_agg(D):
    mesh = plsc.VectorSubcoreMesh(
        core_axis_name="c", subcore_axis_name="s",
        num_cores=NC, num_subcores=NS)

    @functools.partial(
        pl.kernel,
        out_type=jax.ShapeDtypeStruct((NC, NPAD, D), jnp.float32),
        mesh=mesh,
        compiler_params=pltpu.CompilerParams(use_tc_tiling_on_sc=False),
        scratch_types=[
            pltpu.VMEM((2, K), jnp.int32),
            pltpu.VMEM((2, K), jnp.int32),
            pltpu.VMEM((K, D), jnp.float32),
            pltpu.VMEM((K, D), jnp.float32),
            pltpu.VMEM((ZROWS, D), jnp.float32),
            pltpu.VMEM_SHARED((NPAD, D), jnp.float32),
            pltpu.SemaphoreType.DMA,
            pltpu.SemaphoreType.DMA,
            pltpu.SemaphoreType.DMA,
            pltpu.SemaphoreType.DMA,
        ],
    )
    def agg(table_hbm, edge_hbm, out_hbm,
            src_v, dst_v, rows_a, rows_b, zb_v, acc,
            sia, sib, sga, sgb):
        c = lax.axis_index("c")
        s = lax.axis_index("s")
        wid = c * NS + s

        zeros16 = jnp.zeros((16,), jnp.float32)

        def zrow(j, _):
            def zlane(k, _):
                zb_v[j, pl.ds(k * 16, 16)] = zeros16
                return None
            return lax.fori_loop(0, D // 16, zlane, None)
        lax.fori_loop(0, ZROWS, zrow, None)

        def zcopy(j, _):
            pltpu.sync_copy(zb_v, acc.at[pl.ds(s * RPT + j * ZROWS, ZROWS)])
            return None
        lax.fori_loop(0, RPT // ZROWS, zcopy, None)
        plsc.subcore_barrier()

        base = wid * EPW
        last = base + (ITERS - 1) * K

        def issue_idx(i, buf, sem):
            off = pl.multiple_of(
                jnp.minimum(base + i * K, last).astype(jnp.int32), 8)
            pltpu.async_copy(edge_hbm.at[0, pl.ds(off, K)], src_v.at[buf],
                             sem)
            pltpu.async_copy(edge_hbm.at[1, pl.ds(off, K)], dst_v.at[buf],
                             sem)

        def wait_idx(buf, sem):
            pltpu.make_async_copy(edge_hbm.at[0, pl.ds(0, K)],
                                  src_v.at[buf], sem).wait()
            pltpu.make_async_copy(edge_hbm.at[1, pl.ds(0, K)],
                                  dst_v.at[buf], sem).wait()

        issue_idx(0, 0, sia)
        issue_idx(1, 1, sib)
        wait_idx(0, sia)
        pltpu.async_copy(table_hbm.at[src_v.at[0]], rows_a, sga)

        def step(j, _):
            i0 = 2 * j
            wait_idx(1, sib)
            pltpu.async_copy(table_hbm.at[src_v.at[1]], rows_b, sgb)
            pltpu.make_async_copy(table_hbm.at[src_v.at[0]], rows_a,
                                  sga).wait()
            pltpu.sync_copy(rows_a, acc.at[dst_v.at[0]], add=True)
            issue_idx(i0 + 2, 0, sia)
            wait_idx(0, sia)
            pltpu.async_copy(table_hbm.at[src_v.at[0]], rows_a, sga)
            pltpu.make_async_copy(table_hbm.at[src_v.at[1]], rows_b,
                                  sgb).wait()
            pltpu.sync_copy(rows_b, acc.at[dst_v.at[1]], add=True)
            issue_idx(i0 + 3, 1, sib)
            return None
        lax.fori_loop(0, (ITERS - 1) // 2, step, None)

        pltpu.make_async_copy(table_hbm.at[src_v.at[0]], rows_a, sga).wait()
        pltpu.sync_copy(rows_a, acc.at[dst_v.at[0]], add=True)
        wait_idx(1, sib)
        plsc.subcore_barrier()

        pltpu.sync_copy(acc.at[pl.ds(s * RPT, RPT)],
                        out_hbm.at[c, pl.ds(s * RPT, RPT)])

    return agg


def _deg_body(dst_ref, dinv_ref):
    i = pl.program_id(0)
    d = dst_ref[0]
    qi = lax.broadcasted_iota(jnp.int32, (128, EB), 0)
    hiT = (lax.shift_right_logical(d, 7) == qi).astype(jnp.bfloat16)
    loT = ((d & 127) == qi).astype(jnp.bfloat16)
    cnt = lax.dot_general(hiT, loT, (((1,), (1,)), ((), ())),
                          preferred_element_type=jnp.float32)

    @pl.when(i == 0)
    def _():
        dinv_ref[...] = jnp.zeros_like(dinv_ref)

    dinv_ref[...] += cnt

    @pl.when(i == NEB - 1)
    def _():
        dinv_ref[...] = lax.rsqrt(dinv_ref[...] + 1.0)


def _deg_dinv(dst3):
    return pl.pallas_call(
        _deg_body,
        grid=(NEB,),
        in_specs=[pl.BlockSpec((1, 1, EB), lambda i: (i, 0, 0))],
        out_specs=pl.BlockSpec((128, 128), lambda i: (0, 0)),
        out_shape=jax.ShapeDtypeStruct((128, 128), jnp.float32),
    )(dst3)


def _tc1_body(x_ref, g_ref, b_ref, dv_ref, w_ref, hp_ref, s_ref):
    xb = x_ref[...]
    mu = jnp.mean(xb, axis=1, keepdims=True)
    xc = xb - mu
    var = jnp.mean(xc * xc, axis=1, keepdims=True)
    xn = xc * lax.rsqrt(var + EPS) * g_ref[...] + b_ref[...]
    dinv = dv_ref[...]
    hp = jnp.dot(xn * dinv, w_ref[...], preferred_element_type=jnp.float32)
    hp_ref[...] = hp
    s_ref[...] = hp * dinv


def _mid_body(ra_ref, rb_ref, sin_ref, bias_ref, dv_ref, w_ref,
              hp_ref, s_ref):
    dinv = dv_ref[...]
    u = jnp.maximum(
        dinv * (ra_ref[0] + rb_ref[0]) + sin_ref[...] + bias_ref[...],
        0.0)
    hp = jnp.dot(u * dinv, w_ref[...], preferred_element_type=jnp.float32)
    hp_ref[...] = hp
    s_ref[...] = hp * dinv


def _fin_body(ra_ref, rb_ref, sin_ref, bias_ref, dv_ref, out_ref):
    dinv = dv_ref[...]
    out_ref[...] = (dinv * (ra_ref[0] + rb_ref[0])
                    + sin_ref[...] + bias_ref[...])


def _row_spec(d):
    return pl.BlockSpec((B, d), lambda i: (i, 0))


def _full_spec(shape):
    return pl.BlockSpec(shape, lambda i: (0,) * len(shape))


def _tc1(x, g2, b2, dinv, W1):
    return pl.pallas_call(
        _tc1_body,
        grid=(N // B,),
        in_specs=[_row_spec(128), _full_spec((1, 128)), _full_spec((1, 128)),
                  _row_spec(1), _full_spec((128, 128))],
        out_specs=[_row_spec(128), _row_spec(128)],
        out_shape=[jax.ShapeDtypeStruct((N, 128), jnp.float32),
                   jax.ShapeDtypeStruct((N, 128), jnp.float32)],
    )(x, g2, b2, dinv, W1)


def _part_spec(d, c):
    if c == 0:
        return pl.BlockSpec((1, B, d), lambda i: (0, i, 0))
    return pl.BlockSpec((1, B, d), lambda i: (1, i, 0))


def _tc_mid(r, sin, bias2, dinv, W, din, dout):
    return pl.pallas_call(
        _mid_body,
        grid=(N // B,),
        in_specs=[_part_spec(din, 0), _part_spec(din, 1), _row_spec(din),
                  _full_spec((1, din)), _row_spec(1),
                  _full_spec((din, dout))],
        out_specs=[_row_spec(dout), _row_spec(dout)],
        out_shape=[jax.ShapeDtypeStruct((N, dout), jnp.float32),
                   jax.ShapeDtypeStruct((N, dout), jnp.float32)],
    )(r, r, sin, bias2, dinv, W)


def _tc_fin(r, sin, bias2, dinv):
    return pl.pallas_call(
        _fin_body,
        grid=(N // B,),
        in_specs=[_part_spec(32, 0), _part_spec(32, 1), _row_spec(32),
                  _full_spec((1, 32)), _row_spec(1)],
        out_specs=pl.BlockSpec((B, 32), lambda i: (i, 0)),
        out_shape=jax.ShapeDtypeStruct((N, 32), jnp.float32),
    )(r, r, sin, bias2, dinv)


def kernel(x, edge, ln_g, ln_b, W1, b1, W2, b2, W3, b3):
    g2 = ln_g.reshape(1, 128)
    lb2 = ln_b.reshape(1, 128)
    b1_2 = b1.reshape(1, 128)
    b2_2 = b2.reshape(1, 128)
    b3_2 = b3.reshape(1, 32)

    dinv2d = _deg_dinv(edge[1].reshape(NEB, 1, EB))
    dinv = dinv2d.reshape(128 * 128, 1)[:N]

    h1p, s1 = _tc1(x, g2, lb2, dinv, W1)
    r1 = _make_agg(128)(h1p, edge)
    h2p, s2 = _tc_mid(r1, s1, b1_2, dinv, W2, 128, 128)
    r2 = _make_agg(128)(h2p, edge)
    h3p, s3 = _tc_mid(r2, s2, b2_2, dinv, W3, 128, 32)
    r3 = _make_agg(32)(h3p, edge)
    return _tc_fin(r3, s3, b3_2, dinv)


# speedup vs baseline: 22.1223x; 1.1150x over previous
import functools

import jax
import jax.numpy as jnp
from jax import lax
from jax.experimental import pallas as pl
from jax.experimental.pallas import tpu as pltpu
from jax.experimental.pallas import tpu_sc as plsc

N = 10000
NPAD = 10240
E = 320000
NC = 2
NS = 16
NW = NC * NS
EPW = E // NW
K = 80
ITERS = EPW // K
RPT = NPAD // NS
ZROWS = 128
B = 2000
EB = 2000
NEB = E // EB
EPS = 1e-5


@functools.lru_cache(maxsize=None)
def _make_agg(D):
    mesh = plsc.VectorSubcoreMesh(
        core_axis_name="c", subcore_axis_name="s",
        num_cores=NC, num_subcores=NS)

    @functools.partial(
        pl.kernel,
        out_type=jax.ShapeDtypeStruct((NC, NPAD, D), jnp.float32),
        mesh=mesh,
        compiler_params=pltpu.CompilerParams(use_tc_tiling_on_sc=False),
        scratch_types=[
            pltpu.VMEM((2, K), jnp.int32),
            pltpu.VMEM((2, K), jnp.int32),
            pltpu.VMEM((K, D), jnp.float32),
            pltpu.VMEM((K, D), jnp.float32),
            pltpu.VMEM((ZROWS, D), jnp.float32),
            pltpu.VMEM_SHARED((NPAD, D), jnp.float32),
            pltpu.SemaphoreType.DMA,
            pltpu.SemaphoreType.DMA,
            pltpu.SemaphoreType.DMA,
            pltpu.SemaphoreType.DMA,
        ],
    )
    def agg(table_hbm, edge_hbm, out_hbm,
            src_v, dst_v, rows_a, rows_b, zb_v, acc,
            sia, sib, sga, sgb):
        c = lax.axis_index("c")
        s = lax.axis_index("s")
        wid = c * NS + s

        zeros16 = jnp.zeros((16,), jnp.float32)

        def zrow(j, _):
            def zlane(k, _):
                zb_v[j, pl.ds(k * 16, 16)] = zeros16
                return None
            return lax.fori_loop(0, D // 16, zlane, None)
        lax.fori_loop(0, ZROWS, zrow, None)

        def zcopy(j, _):
            pltpu.sync_copy(zb_v, acc.at[pl.ds(s * RPT + j * ZROWS, ZROWS)])
            return None
        lax.fori_loop(0, RPT // ZROWS, zcopy, None)
        plsc.subcore_barrier()

        base = wid * EPW
        last = base + (ITERS - 1) * K

        def issue_idx(i, buf, sem):
            off = pl.multiple_of(
                jnp.minimum(base + i * K, last).astype(jnp.int32), 8)
            pltpu.async_copy(edge_hbm.at[0, pl.ds(off, K)], src_v.at[buf],
                             sem)
            pltpu.async_copy(edge_hbm.at[1, pl.ds(off, K)], dst_v.at[buf],
                             sem)

        def wait_idx(buf, sem):
            pltpu.make_async_copy(edge_hbm.at[0, pl.ds(0, K)],
                                  src_v.at[buf], sem).wait()
            pltpu.make_async_copy(edge_hbm.at[1, pl.ds(0, K)],
                                  dst_v.at[buf], sem).wait()

        issue_idx(0, 0, sia)
        issue_idx(1, 1, sib)
        wait_idx(0, sia)
        pltpu.async_copy(table_hbm.at[src_v.at[0]], rows_a, sga)

        def step(j, _):
            i0 = 2 * j
            wait_idx(1, sib)
            pltpu.async_copy(table_hbm.at[src_v.at[1]], rows_b, sgb)
            pltpu.make_async_copy(table_hbm.at[src_v.at[0]], rows_a,
                                  sga).wait()
            pltpu.sync_copy(rows_a, acc.at[dst_v.at[0]], add=True)
            issue_idx(i0 + 2, 0, sia)
            wait_idx(0, sia)
            pltpu.async_copy(table_hbm.at[src_v.at[0]], rows_a, sga)
            pltpu.make_async_copy(table_hbm.at[src_v.at[1]], rows_b,
                                  sgb).wait()
            pltpu.sync_copy(rows_b, acc.at[dst_v.at[1]], add=True)
            issue_idx(i0 + 3, 1, sib)
            return None
        lax.fori_loop(0, (ITERS - 1) // 2, step, None)

        pltpu.make_async_copy(table_hbm.at[src_v.at[0]], rows_a, sga).wait()
        pltpu.sync_copy(rows_a, acc.at[dst_v.at[0]], add=True)
        wait_idx(1, sib)
        plsc.subcore_barrier()

        pltpu.sync_copy(acc.at[pl.ds(s * RPT, RPT)],
                        out_hbm.at[c, pl.ds(s * RPT, RPT)])

    return agg


def _make_deg():
    mesh = plsc.VectorSubcoreMesh(
        core_axis_name="c", subcore_axis_name="s",
        num_cores=NC, num_subcores=NS)

    @functools.partial(
        pl.kernel,
        out_type=jax.ShapeDtypeStruct((NC, NPAD, 16), jnp.float32),
        mesh=mesh,
        compiler_params=pltpu.CompilerParams(use_tc_tiling_on_sc=False),
        scratch_types=[
            pltpu.VMEM((2, K), jnp.int32),
            pltpu.VMEM((K, 16), jnp.float32),
            pltpu.VMEM((ZROWS, 16), jnp.float32),
            pltpu.VMEM_SHARED((NPAD, 16), jnp.float32),
            pltpu.SemaphoreType.DMA,
            pltpu.SemaphoreType.DMA,
        ],
    )
    def deg(edge_hbm, out_hbm, dst_v, ones_v, zb_v, acc, sia, sib):
        c = lax.axis_index("c")
        s = lax.axis_index("s")
        wid = c * NS + s

        ones16 = jnp.ones((16,), jnp.float32)
        zeros16 = jnp.zeros((16,), jnp.float32)

        def fill(j, _):
            ones_v[j, :] = ones16
            return None
        lax.fori_loop(0, K, fill, None)

        def zrow(j, _):
            zb_v[j, :] = zeros16
            return None
        lax.fori_loop(0, ZROWS, zrow, None)

        def zcopy(j, _):
            pltpu.sync_copy(zb_v, acc.at[pl.ds(s * RPT + j * ZROWS, ZROWS)])
            return None
        lax.fori_loop(0, RPT // ZROWS, zcopy, None)
        plsc.subcore_barrier()

        base = wid * EPW
        last = base + (ITERS - 1) * K

        def issue_idx(i, buf, sem):
            off = pl.multiple_of(
                jnp.minimum(base + i * K, last).astype(jnp.int32), 8)
            pltpu.async_copy(edge_hbm.at[1, pl.ds(off, K)], dst_v.at[buf],
                             sem)

        def wait_idx(buf, sem):
            pltpu.make_async_copy(edge_hbm.at[1, pl.ds(0, K)],
                                  dst_v.at[buf], sem).wait()

        issue_idx(0, 0, sia)
        issue_idx(1, 1, sib)

        def step(j, _):
            i0 = 2 * j
            wait_idx(0, sia)
            pltpu.sync_copy(ones_v, acc.at[dst_v.at[0]], add=True)
            issue_idx(i0 + 2, 0, sia)
            wait_idx(1, sib)
            pltpu.sync_copy(ones_v, acc.at[dst_v.at[1]], add=True)
            issue_idx(i0 + 3, 1, sib)
            return None
        lax.fori_loop(0, (ITERS - 1) // 2, step, None)

        wait_idx(0, sia)
        pltpu.sync_copy(ones_v, acc.at[dst_v.at[0]], add=True)
        wait_idx(1, sib)
        plsc.subcore_barrier()

        pltpu.sync_copy(acc.at[pl.ds(s * RPT, RPT)],
                        out_hbm.at[c, pl.ds(s * RPT, RPT)])

    return deg


def _dinv_of(da_ref, db_ref):
    return lax.rsqrt(da_ref[0][:, :1] + db_ref[0][:, :1] + 1.0)


def _tc1_body(x_ref, g_ref, b_ref, w_ref, p_ref):
    xb = x_ref[...]
    mu = jnp.mean(xb, axis=1, keepdims=True)
    xc = xb - mu
    var = jnp.mean(xc * xc, axis=1, keepdims=True)
    xn = xc * lax.rsqrt(var + EPS) * g_ref[...] + b_ref[...]
    p_ref[...] = jnp.dot(xn, w_ref[...], preferred_element_type=jnp.float32)


def _tc1b_body(p_ref, da_ref, db_ref, hp_ref, s_ref):
    dinv = _dinv_of(da_ref, db_ref)
    hp = p_ref[...] * dinv
    hp_ref[...] = hp
    s_ref[...] = hp * dinv


def _mid_body(ra_ref, rb_ref, sin_ref, bias_ref, da_ref, db_ref, w_ref,
              hp_ref, s_ref):
    dinv = _dinv_of(da_ref, db_ref)
    u = jnp.maximum(
        dinv * (ra_ref[0] + rb_ref[0]) + sin_ref[...] + bias_ref[...],
        0.0)
    hp = jnp.dot(u * dinv, w_ref[...], preferred_element_type=jnp.float32)
    hp_ref[...] = hp
    s_ref[...] = hp * dinv


def _fin_body(ra_ref, rb_ref, sin_ref, bias_ref, da_ref, db_ref, out_ref):
    dinv = _dinv_of(da_ref, db_ref)
    out_ref[...] = (dinv * (ra_ref[0] + rb_ref[0])
                    + sin_ref[...] + bias_ref[...])


def _row_spec(d):
    return pl.BlockSpec((B, d), lambda i: (i, 0))


def _full_spec(shape):
    return pl.BlockSpec(shape, lambda i: (0,) * len(shape))


def _tc1(x, g2, b2, W1):
    return pl.pallas_call(
        _tc1_body,
        grid=(N // B,),
        in_specs=[_row_spec(128), _full_spec((1, 128)), _full_spec((1, 128)),
                  _full_spec((128, 128))],
        out_specs=_row_spec(128),
        out_shape=jax.ShapeDtypeStruct((N, 128), jnp.float32),
    )(x, g2, b2, W1)


def _tc1b(p1, deg):
    return pl.pallas_call(
        _tc1b_body,
        grid=(N // B,),
        in_specs=[_row_spec(128), _part_spec(16, 0), _part_spec(16, 1)],
        out_specs=[_row_spec(128), _row_spec(128)],
        out_shape=[jax.ShapeDtypeStruct((N, 128), jnp.float32),
                   jax.ShapeDtypeStruct((N, 128), jnp.float32)],
    )(p1, deg, deg)


def _part_spec(d, c):
    if c == 0:
        return pl.BlockSpec((1, B, d), lambda i: (0, i, 0))
    return pl.BlockSpec((1, B, d), lambda i: (1, i, 0))


def _tc_mid(r, sin, bias2, deg, W, din, dout):
    return pl.pallas_call(
        _mid_body,
        grid=(N // B,),
        in_specs=[_part_spec(din, 0), _part_spec(din, 1), _row_spec(din),
                  _full_spec((1, din)), _part_spec(16, 0), _part_spec(16, 1),
                  _full_spec((din, dout))],
        out_specs=[_row_spec(dout), _row_spec(dout)],
        out_shape=[jax.ShapeDtypeStruct((N, dout), jnp.float32),
                   jax.ShapeDtypeStruct((N, dout), jnp.float32)],
    )(r, r, sin, bias2, deg, deg, W)


def _tc_fin(r, sin, bias2, deg):
    return pl.pallas_call(
        _fin_body,
        grid=(N // B,),
        in_specs=[_part_spec(32, 0), _part_spec(32, 1), _row_spec(32),
                  _full_spec((1, 32)), _part_spec(16, 0), _part_spec(16, 1)],
        out_specs=pl.BlockSpec((B, 32), lambda i: (i, 0)),
        out_shape=jax.ShapeDtypeStruct((N, 32), jnp.float32),
    )(r, r, sin, bias2, deg, deg)


def kernel(x, edge, ln_g, ln_b, W1, b1, W2, b2, W3, b3):
    g2 = ln_g.reshape(1, 128)
    lb2 = ln_b.reshape(1, 128)
    b1_2 = b1.reshape(1, 128)
    b2_2 = b2.reshape(1, 128)
    b3_2 = b3.reshape(1, 32)

    deg = _make_deg()(edge)
    p1 = _tc1(x, g2, lb2, W1)
    h1p, s1 = _tc1b(p1, deg)
    r1 = _make_agg(128)(h1p, edge)
    h2p, s2 = _tc_mid(r1, s1, b1_2, deg, W2, 128, 128)
    r2 = _make_agg(128)(h2p, edge)
    h3p, s3 = _tc_mid(r2, s2, b2_2, deg, W3, 128, 32)
    r3 = _make_agg(32)(h3p, edge)
    return _tc_fin(r3, s3, b3_2, deg)


# speedup vs baseline: 22.1826x; 1.0027x over previous
import functools

import jax
import jax.numpy as jnp
from jax import lax
from jax.experimental import pallas as pl
from jax.experimental.pallas import tpu as pltpu
from jax.experimental.pallas import tpu_sc as plsc

N = 10000
NPAD = 10240
E = 320000
NC = 2
NS = 16
NW = NC * NS
EPW = E // NW
K = 80
ITERS = EPW // K
RPT = NPAD // NS
ZROWS = 128
B = 2000
EB = 2000
NEB = E // EB
EPS = 1e-5


@functools.lru_cache(maxsize=None)
def _make_agg(D):
    mesh = plsc.VectorSubcoreMesh(
        core_axis_name="c", subcore_axis_name="s",
        num_cores=NC, num_subcores=NS)

    @functools.partial(
        pl.kernel,
        out_type=jax.ShapeDtypeStruct((NC, NPAD, D), jnp.float32),
        mesh=mesh,
        compiler_params=pltpu.CompilerParams(use_tc_tiling_on_sc=False),
        scratch_types=[
            pltpu.VMEM((2, K), jnp.int32),
            pltpu.VMEM((2, K), jnp.int32),
            pltpu.VMEM((K, D), jnp.float32),
            pltpu.VMEM((K, D), jnp.float32),
            pltpu.VMEM((ZROWS, D), jnp.float32),
            pltpu.VMEM_SHARED((NPAD, D), jnp.float32),
            pltpu.SemaphoreType.DMA,
            pltpu.SemaphoreType.DMA,
            pltpu.SemaphoreType.DMA,
            pltpu.SemaphoreType.DMA,
        ],
    )
    def agg(table_hbm, edge_hbm, out_hbm,
            src_v, dst_v, rows_a, rows_b, zb_v, acc,
            sia, sib, sga, sgb):
        c = lax.axis_index("c")
        s = lax.axis_index("s")
        wid = c * NS + s

        zeros16 = jnp.zeros((16,), jnp.float32)

        def zrow(j, _):
            def zlane(k, _):
                zb_v[j, pl.ds(k * 16, 16)] = zeros16
                return None
            return lax.fori_loop(0, D // 16, zlane, None)
        lax.fori_loop(0, ZROWS, zrow, None)

        def zcopy(j, _):
            pltpu.sync_copy(zb_v, acc.at[pl.ds(s * RPT + j * ZROWS, ZROWS)])
            return None
        lax.fori_loop(0, RPT // ZROWS, zcopy, None)
        plsc.subcore_barrier()

        base = wid * EPW
        last = base + (ITERS - 1) * K

        def issue_idx(i, buf, sem):
            off = pl.multiple_of(
                jnp.minimum(base + i * K, last).astype(jnp.int32), 8)
            pltpu.async_copy(edge_hbm.at[0, pl.ds(off, K)], src_v.at[buf],
                             sem)
            pltpu.async_copy(edge_hbm.at[1, pl.ds(off, K)], dst_v.at[buf],
                             sem)

        def wait_idx(buf, sem):
            pltpu.make_async_copy(edge_hbm.at[0, pl.ds(0, K)],
                                  src_v.at[buf], sem).wait()
            pltpu.make_async_copy(edge_hbm.at[1, pl.ds(0, K)],
                                  dst_v.at[buf], sem).wait()

        issue_idx(0, 0, sia)
        issue_idx(1, 1, sib)
        wait_idx(0, sia)
        pltpu.async_copy(table_hbm.at[src_v.at[0]], rows_a, sga)

        def step(j, _):
            i0 = 2 * j
            wait_idx(1, sib)
            pltpu.async_copy(table_hbm.at[src_v.at[1]], rows_b, sgb)
            pltpu.make_async_copy(table_hbm.at[src_v.at[0]], rows_a,
                                  sga).wait()
            pltpu.sync_copy(rows_a, acc.at[dst_v.at[0]], add=True)
            issue_idx(i0 + 2, 0, sia)
            wait_idx(0, sia)
            pltpu.async_copy(table_hbm.at[src_v.at[0]], rows_a, sga)
            pltpu.make_async_copy(table_hbm.at[src_v.at[1]], rows_b,
                                  sgb).wait()
            pltpu.sync_copy(rows_b, acc.at[dst_v.at[1]], add=True)
            issue_idx(i0 + 3, 1, sib)
            return None
        lax.fori_loop(0, (ITERS - 1) // 2, step, None)

        pltpu.make_async_copy(table_hbm.at[src_v.at[0]], rows_a, sga).wait()
        pltpu.sync_copy(rows_a, acc.at[dst_v.at[0]], add=True)
        wait_idx(1, sib)
        plsc.subcore_barrier()

        pltpu.sync_copy(acc.at[pl.ds(s * RPT, RPT)],
                        out_hbm.at[c, pl.ds(s * RPT, RPT)])

    return agg


def _make_deg():
    mesh = plsc.VectorSubcoreMesh(
        core_axis_name="c", subcore_axis_name="s",
        num_cores=NC, num_subcores=NS)

    @functools.partial(
        pl.kernel,
        out_type=jax.ShapeDtypeStruct((NC, NPAD, 16), jnp.float32),
        mesh=mesh,
        compiler_params=pltpu.CompilerParams(use_tc_tiling_on_sc=False),
        scratch_types=[
            pltpu.VMEM((2, K), jnp.int32),
            pltpu.VMEM((K, 16), jnp.float32),
            pltpu.VMEM((ZROWS, 16), jnp.float32),
            pltpu.VMEM_SHARED((NPAD, 16), jnp.float32),
            pltpu.SemaphoreType.DMA,
            pltpu.SemaphoreType.DMA,
            pltpu.SemaphoreType.DMA,
            pltpu.SemaphoreType.DMA,
        ],
    )
    def deg(edge_hbm, out_hbm, dst_v, ones_v, zb_v, acc, sia, sib, ssa, ssb):
        c = lax.axis_index("c")
        s = lax.axis_index("s")
        wid = c * NS + s

        ones16 = jnp.ones((16,), jnp.float32)
        zeros16 = jnp.zeros((16,), jnp.float32)

        def fill(j, _):
            ones_v[j, :] = ones16
            return None
        lax.fori_loop(0, K, fill, None)

        def zrow(j, _):
            zb_v[j, :] = zeros16
            return None
        lax.fori_loop(0, ZROWS, zrow, None)

        def zcopy(j, _):
            pltpu.sync_copy(zb_v, acc.at[pl.ds(s * RPT + j * ZROWS, ZROWS)])
            return None
        lax.fori_loop(0, RPT // ZROWS, zcopy, None)
        plsc.subcore_barrier()

        base = wid * EPW
        last = base + (ITERS - 1) * K

        def issue_idx(i, buf, sem):
            off = pl.multiple_of(
                jnp.minimum(base + i * K, last).astype(jnp.int32), 8)
            pltpu.async_copy(edge_hbm.at[1, pl.ds(off, K)], dst_v.at[buf],
                             sem)

        def wait_idx(buf, sem):
            pltpu.make_async_copy(edge_hbm.at[1, pl.ds(0, K)],
                                  dst_v.at[buf], sem).wait()

        issue_idx(0, 0, sia)
        issue_idx(1, 1, sib)

        def step(j, _):
            i0 = 2 * j
            wait_idx(0, sia)
            da = pltpu.async_copy(ones_v, acc.at[dst_v.at[0]], ssa,
                                  add=True)
            wait_idx(1, sib)
            db = pltpu.async_copy(ones_v, acc.at[dst_v.at[1]], ssb,
                                  add=True)
            da.wait()
            issue_idx(i0 + 2, 0, sia)
            db.wait()
            issue_idx(i0 + 3, 1, sib)
            return None
        lax.fori_loop(0, (ITERS - 1) // 2, step, None)

        wait_idx(0, sia)
        pltpu.async_copy(ones_v, acc.at[dst_v.at[0]], ssa, add=True).wait()
        wait_idx(1, sib)
        plsc.subcore_barrier()

        pltpu.sync_copy(acc.at[pl.ds(s * RPT, RPT)],
                        out_hbm.at[c, pl.ds(s * RPT, RPT)])

    return deg


def _dinv_of(da_ref, db_ref):
    return lax.rsqrt(da_ref[0][:, :1] + db_ref[0][:, :1] + 1.0)


def _tc1_body(x_ref, g_ref, b_ref, w_ref, p_ref):
    xb = x_ref[...]
    mu = jnp.mean(xb, axis=1, keepdims=True)
    xc = xb - mu
    var = jnp.mean(xc * xc, axis=1, keepdims=True)
    xn = xc * lax.rsqrt(var + EPS) * g_ref[...] + b_ref[...]
    p_ref[...] = jnp.dot(xn, w_ref[...], preferred_element_type=jnp.float32)


def _tc1b_body(p_ref, da_ref, db_ref, hp_ref, s_ref, dv_ref):
    dinv = _dinv_of(da_ref, db_ref)
    hp = p_ref[...] * dinv
    hp_ref[...] = hp
    s_ref[...] = hp * dinv
    dv_ref[...] = dinv * jnp.ones((1, 16), jnp.float32)


def _mid_body(ra_ref, rb_ref, sin_ref, bias_ref, dv_ref, w_ref,
              hp_ref, s_ref):
    dinv = dv_ref[:, :1]
    u = jnp.maximum(
        dinv * (ra_ref[0] + rb_ref[0]) + sin_ref[...] + bias_ref[...],
        0.0)
    hp = jnp.dot(u * dinv, w_ref[...], preferred_element_type=jnp.float32)
    hp_ref[...] = hp
    s_ref[...] = hp * dinv


def _fin_body(ra_ref, rb_ref, sin_ref, bias_ref, dv_ref, out_ref):
    dinv = dv_ref[:, :1]
    out_ref[...] = (dinv * (ra_ref[0] + rb_ref[0])
                    + sin_ref[...] + bias_ref[...])


def _row_spec(d):
    return pl.BlockSpec((B, d), lambda i: (i, 0))


def _full_spec(shape):
    return pl.BlockSpec(shape, lambda i: (0,) * len(shape))


def _tc1(x, g2, b2, W1):
    return pl.pallas_call(
        _tc1_body,
        grid=(N // B,),
        in_specs=[_row_spec(128), _full_spec((1, 128)), _full_spec((1, 128)),
                  _full_spec((128, 128))],
        out_specs=_row_spec(128),
        out_shape=jax.ShapeDtypeStruct((N, 128), jnp.float32),
    )(x, g2, b2, W1)


def _tc1b(p1, deg):
    return pl.pallas_call(
        _tc1b_body,
        grid=(N // B,),
        in_specs=[_row_spec(128), _part_spec(16, 0), _part_spec(16, 1)],
        out_specs=[_row_spec(128), _row_spec(128), _row_spec(16)],
        out_shape=[jax.ShapeDtypeStruct((N, 128), jnp.float32),
                   jax.ShapeDtypeStruct((N, 128), jnp.float32),
                   jax.ShapeDtypeStruct((N, 16), jnp.float32)],
    )(p1, deg, deg)


def _part_spec(d, c):
    if c == 0:
        return pl.BlockSpec((1, B, d), lambda i: (0, i, 0))
    return pl.BlockSpec((1, B, d), lambda i: (1, i, 0))


def _tc_mid(r, sin, bias2, dinv16, W, din, dout):
    return pl.pallas_call(
        _mid_body,
        grid=(N // B,),
        in_specs=[_part_spec(din, 0), _part_spec(din, 1), _row_spec(din),
                  _full_spec((1, din)), _row_spec(16),
                  _full_spec((din, dout))],
        out_specs=[_row_spec(dout), _row_spec(dout)],
        out_shape=[jax.ShapeDtypeStruct((N, dout), jnp.float32),
                   jax.ShapeDtypeStruct((N, dout), jnp.float32)],
    )(r, r, sin, bias2, dinv16, W)


def _tc_fin(r, sin, bias2, dinv16):
    return pl.pallas_call(
        _fin_body,
        grid=(N // B,),
        in_specs=[_part_spec(32, 0), _part_spec(32, 1), _row_spec(32),
                  _full_spec((1, 32)), _row_spec(16)],
        out_specs=pl.BlockSpec((B, 32), lambda i: (i, 0)),
        out_shape=jax.ShapeDtypeStruct((N, 32), jnp.float32),
    )(r, r, sin, bias2, dinv16)


def kernel(x, edge, ln_g, ln_b, W1, b1, W2, b2, W3, b3):
    g2 = ln_g.reshape(1, 128)
    lb2 = ln_b.reshape(1, 128)
    b1_2 = b1.reshape(1, 128)
    b2_2 = b2.reshape(1, 128)
    b3_2 = b3.reshape(1, 32)

    deg = _make_deg()(edge)
    p1 = _tc1(x, g2, lb2, W1)
    h1p, s1, dinv16 = _tc1b(p1, deg)
    r1 = _make_agg(128)(h1p, edge)
    h2p, s2 = _tc_mid(r1, s1, b1_2, dinv16, W2, 128, 128)
    r2 = _make_agg(128)(h2p, edge)
    h3p, s3 = _tc_mid(r2, s2, b2_2, dinv16, W3, 128, 32)
    r3 = _make_agg(32)(h3p, edge)
    return _tc_fin(r3, s3, b3_2, dinv16)


# speedup vs baseline: 25.9081x; 1.1679x over previous
import functools

import jax
import jax.numpy as jnp
from jax import lax
from jax.experimental import pallas as pl
from jax.experimental.pallas import tpu as pltpu
from jax.experimental.pallas import tpu_sc as plsc

N = 10000
NPAD = 10240
E = 320000
NC = 2
NS = 16
NW = NC * NS
EPW = E // NW
K = 128
FULL = EPW // K
TAIL = EPW - FULL * K
RPT = NPAD // NS
ZROWS = 64
B = 2000
EB = 2000
NEB = E // EB
EPS = 1e-5


@functools.lru_cache(maxsize=None)
def _make_agg(D):
    mesh = plsc.VectorSubcoreMesh(
        core_axis_name="c", subcore_axis_name="s",
        num_cores=NC, num_subcores=NS)

    @functools.partial(
        pl.kernel,
        out_type=jax.ShapeDtypeStruct((NC, NPAD, D), jnp.float32),
        mesh=mesh,
        compiler_params=pltpu.CompilerParams(use_tc_tiling_on_sc=False),
        scratch_types=[
            pltpu.VMEM((2, K), jnp.int32),
            pltpu.VMEM((2, K), jnp.int32),
            pltpu.VMEM((K, D), jnp.float32),
            pltpu.VMEM((K, D), jnp.float32),
            pltpu.VMEM((TAIL,), jnp.int32),
            pltpu.VMEM((TAIL,), jnp.int32),
            pltpu.VMEM((TAIL, D), jnp.float32),
            pltpu.VMEM((ZROWS, D), jnp.float32),
            pltpu.VMEM_SHARED((NPAD, D), jnp.float32),
            pltpu.SemaphoreType.DMA,
            pltpu.SemaphoreType.DMA,
            pltpu.SemaphoreType.DMA,
            pltpu.SemaphoreType.DMA,
        ],
    )
    def agg(table_hbm, edge_hbm, out_hbm,
            src_v, dst_v, rows_a, rows_b, st_v, dt_v, rows_t, zb_v, acc,
            sia, sib, sga, sgb):
        c = lax.axis_index("c")
        s = lax.axis_index("s")
        wid = c * NS + s

        zeros16 = jnp.zeros((16,), jnp.float32)

        def zrow(j, _):
            def zlane(k, _):
                zb_v[j, pl.ds(k * 16, 16)] = zeros16
                return None
            return lax.fori_loop(0, D // 16, zlane, None)
        lax.fori_loop(0, ZROWS, zrow, None)

        def zcopy(j, _):
            pltpu.sync_copy(zb_v, acc.at[pl.ds(s * RPT + j * ZROWS, ZROWS)])
            return None
        lax.fori_loop(0, RPT // ZROWS, zcopy, None)
        plsc.subcore_barrier()

        base = wid * EPW
        last = base + (FULL - 1) * K

        def issue_idx(i, buf, sem):
            off = pl.multiple_of(
                jnp.minimum(base + i * K, last).astype(jnp.int32), 8)
            pltpu.async_copy(edge_hbm.at[0, pl.ds(off, K)], src_v.at[buf],
                             sem)
            pltpu.async_copy(edge_hbm.at[1, pl.ds(off, K)], dst_v.at[buf],
                             sem)

        def wait_idx(buf, sem):
            pltpu.make_async_copy(edge_hbm.at[0, pl.ds(0, K)],
                                  src_v.at[buf], sem).wait()
            pltpu.make_async_copy(edge_hbm.at[1, pl.ds(0, K)],
                                  dst_v.at[buf], sem).wait()

        issue_idx(0, 0, sia)
        issue_idx(1, 1, sib)
        wait_idx(0, sia)
        pltpu.async_copy(table_hbm.at[src_v.at[0]], rows_a, sga)

        def step(j, _):
            i0 = 2 * j
            wait_idx(1, sib)
            pltpu.async_copy(table_hbm.at[src_v.at[1]], rows_b, sgb)
            pltpu.make_async_copy(table_hbm.at[src_v.at[0]], rows_a,
                                  sga).wait()
            pltpu.sync_copy(rows_a, acc.at[dst_v.at[0]], add=True)
            issue_idx(i0 + 2, 0, sia)
            wait_idx(0, sia)
            pltpu.async_copy(table_hbm.at[src_v.at[0]], rows_a, sga)
            pltpu.make_async_copy(table_hbm.at[src_v.at[1]], rows_b,
                                  sgb).wait()
            pltpu.sync_copy(rows_b, acc.at[dst_v.at[1]], add=True)
            issue_idx(i0 + 3, 1, sib)
            return None
        lax.fori_loop(0, FULL // 2, step, None)

        pltpu.make_async_copy(table_hbm.at[src_v.at[0]], rows_a, sga).wait()
        if FULL % 2:
            pltpu.sync_copy(rows_a, acc.at[dst_v.at[0]], add=True)
        wait_idx(1, sib)
        if TAIL:
            toff = pl.multiple_of(base + FULL * K, 8)
            pltpu.sync_copy(edge_hbm.at[0, pl.ds(toff, TAIL)], st_v)
            pltpu.sync_copy(edge_hbm.at[1, pl.ds(toff, TAIL)], dt_v)
            pltpu.async_copy(table_hbm.at[st_v], rows_t, sga).wait()
            pltpu.sync_copy(rows_t, acc.at[dt_v], add=True)
        plsc.subcore_barrier()

        pltpu.sync_copy(acc.at[pl.ds(s * RPT, RPT)],
                        out_hbm.at[c, pl.ds(s * RPT, RPT)])

    return agg


def _make_deg():
    mesh = plsc.VectorSubcoreMesh(
        core_axis_name="c", subcore_axis_name="s",
        num_cores=NC, num_subcores=NS)

    @functools.partial(
        pl.kernel,
        out_type=jax.ShapeDtypeStruct((NC, NPAD, 16), jnp.float32),
        mesh=mesh,
        compiler_params=pltpu.CompilerParams(use_tc_tiling_on_sc=False),
        scratch_types=[
            pltpu.VMEM((2, K), jnp.int32),
            pltpu.VMEM((TAIL,), jnp.int32),
            pltpu.VMEM((K, 16), jnp.float32),
            pltpu.VMEM((ZROWS, 16), jnp.float32),
            pltpu.VMEM_SHARED((NPAD, 16), jnp.float32),
            pltpu.SemaphoreType.DMA,
            pltpu.SemaphoreType.DMA,
            pltpu.SemaphoreType.DMA,
            pltpu.SemaphoreType.DMA,
        ],
    )
    def deg(edge_hbm, out_hbm, dst_v, dt_v, ones_v, zb_v, acc,
            sia, sib, ssa, ssb):
        c = lax.axis_index("c")
        s = lax.axis_index("s")
        wid = c * NS + s

        ones16 = jnp.ones((16,), jnp.float32)
        zeros16 = jnp.zeros((16,), jnp.float32)

        def fill(j, _):
            ones_v[j, :] = ones16
            return None
        lax.fori_loop(0, K, fill, None)

        def zrow(j, _):
            zb_v[j, :] = zeros16
            return None
        lax.fori_loop(0, ZROWS, zrow, None)

        def zcopy(j, _):
            pltpu.sync_copy(zb_v, acc.at[pl.ds(s * RPT + j * ZROWS, ZROWS)])
            return None
        lax.fori_loop(0, RPT // ZROWS, zcopy, None)
        plsc.subcore_barrier()

        base = wid * EPW
        last = base + (FULL - 1) * K

        def issue_idx(i, buf, sem):
            off = pl.multiple_of(
                jnp.minimum(base + i * K, last).astype(jnp.int32), 8)
            pltpu.async_copy(edge_hbm.at[1, pl.ds(off, K)], dst_v.at[buf],
                             sem)

        def wait_idx(buf, sem):
            pltpu.make_async_copy(edge_hbm.at[1, pl.ds(0, K)],
                                  dst_v.at[buf], sem).wait()

        issue_idx(0, 0, sia)
        issue_idx(1, 1, sib)

        def step(j, _):
            i0 = 2 * j
            wait_idx(0, sia)
            da = pltpu.async_copy(ones_v, acc.at[dst_v.at[0]], ssa,
                                  add=True)
            wait_idx(1, sib)
            db = pltpu.async_copy(ones_v, acc.at[dst_v.at[1]], ssb,
                                  add=True)
            da.wait()
            issue_idx(i0 + 2, 0, sia)
            db.wait()
            issue_idx(i0 + 3, 1, sib)
            return None
        lax.fori_loop(0, FULL // 2, step, None)

        wait_idx(0, sia)
        if FULL % 2:
            pltpu.async_copy(ones_v, acc.at[dst_v.at[0]], ssa,
                             add=True).wait()
        wait_idx(1, sib)
        if TAIL:
            toff = pl.multiple_of(base + FULL * K, 8)
            pltpu.sync_copy(edge_hbm.at[1, pl.ds(toff, TAIL)], dt_v)
            pltpu.async_copy(ones_v.at[pl.ds(0, TAIL)], acc.at[dt_v], ssa,
                             add=True).wait()
        plsc.subcore_barrier()

        pltpu.sync_copy(acc.at[pl.ds(s * RPT, RPT)],
                        out_hbm.at[c, pl.ds(s * RPT, RPT)])

    return deg


def _dinv_of(da_ref, db_ref):
    return lax.rsqrt(da_ref[0][:, :1] + db_ref[0][:, :1] + 1.0)


def _tc1_body(x_ref, g_ref, b_ref, w_ref, p_ref):
    xb = x_ref[...]
    mu = jnp.mean(xb, axis=1, keepdims=True)
    xc = xb - mu
    var = jnp.mean(xc * xc, axis=1, keepdims=True)
    xn = xc * lax.rsqrt(var + EPS) * g_ref[...] + b_ref[...]
    p_ref[...] = jnp.dot(xn, w_ref[...], preferred_element_type=jnp.float32)


def _tc1b_body(p_ref, da_ref, db_ref, hp_ref, s_ref, dv_ref):
    dinv = _dinv_of(da_ref, db_ref)
    hp = p_ref[...] * dinv
    hp_ref[...] = hp
    s_ref[...] = hp * dinv
    dv_ref[...] = dinv * jnp.ones((1, 16), jnp.float32)


def _mid_body(ra_ref, rb_ref, sin_ref, bias_ref, dv_ref, w_ref,
              hp_ref, s_ref):
    dinv = dv_ref[:, :1]
    u = jnp.maximum(
        dinv * (ra_ref[0] + rb_ref[0]) + sin_ref[...] + bias_ref[...],
        0.0)
    hp = jnp.dot(u * dinv, w_ref[...], preferred_element_type=jnp.float32)
    hp_ref[...] = hp
    s_ref[...] = hp * dinv


def _fin_body(ra_ref, rb_ref, sin_ref, bias_ref, dv_ref, out_ref):
    dinv = dv_ref[:, :1]
    out_ref[...] = (dinv * (ra_ref[0] + rb_ref[0])
                    + sin_ref[...] + bias_ref[...])


def _row_spec(d):
    return pl.BlockSpec((B, d), lambda i: (i, 0))


def _full_spec(shape):
    return pl.BlockSpec(shape, lambda i: (0,) * len(shape))


def _tc1(x, g2, b2, W1):
    return pl.pallas_call(
        _tc1_body,
        grid=(N // B,),
        in_specs=[_row_spec(128), _full_spec((1, 128)), _full_spec((1, 128)),
                  _full_spec((128, 128))],
        out_specs=_row_spec(128),
        out_shape=jax.ShapeDtypeStruct((N, 128), jnp.float32),
    )(x, g2, b2, W1)


def _tc1b(p1, deg):
    return pl.pallas_call(
        _tc1b_body,
        grid=(N // B,),
        in_specs=[_row_spec(128), _part_spec(16, 0), _part_spec(16, 1)],
        out_specs=[_row_spec(128), _row_spec(128), _row_spec(16)],
        out_shape=[jax.ShapeDtypeStruct((N, 128), jnp.float32),
                   jax.ShapeDtypeStruct((N, 128), jnp.float32),
                   jax.ShapeDtypeStruct((N, 16), jnp.float32)],
    )(p1, deg, deg)


def _part_spec(d, c):
    if c == 0:
        return pl.BlockSpec((1, B, d), lambda i: (0, i, 0))
    return pl.BlockSpec((1, B, d), lambda i: (1, i, 0))


def _tc_mid(r, sin, bias2, dinv16, W, din, dout):
    return pl.pallas_call(
        _mid_body,
        grid=(N // B,),
        in_specs=[_part_spec(din, 0), _part_spec(din, 1), _row_spec(din),
                  _full_spec((1, din)), _row_spec(16),
                  _full_spec((din, dout))],
        out_specs=[_row_spec(dout), _row_spec(dout)],
        out_shape=[jax.ShapeDtypeStruct((N, dout), jnp.float32),
                   jax.ShapeDtypeStruct((N, dout), jnp.float32)],
    )(r, r, sin, bias2, dinv16, W)


def _tc_fin(r, sin, bias2, dinv16):
    return pl.pallas_call(
        _fin_body,
        grid=(N // B,),
        in_specs=[_part_spec(32, 0), _part_spec(32, 1), _row_spec(32),
                  _full_spec((1, 32)), _row_spec(16)],
        out_specs=pl.BlockSpec((B, 32), lambda i: (i, 0)),
        out_shape=jax.ShapeDtypeStruct((N, 32), jnp.float32),
    )(r, r, sin, bias2, dinv16)


def kernel(x, edge, ln_g, ln_b, W1, b1, W2, b2, W3, b3):
    g2 = ln_g.reshape(1, 128)
    lb2 = ln_b.reshape(1, 128)
    b1_2 = b1.reshape(1, 128)
    b2_2 = b2.reshape(1, 128)
    b3_2 = b3.reshape(1, 32)

    deg = _make_deg()(edge)
    p1 = _tc1(x, g2, lb2, W1)
    h1p, s1, dinv16 = _tc1b(p1, deg)
    r1 = _make_agg(128)(h1p, edge)
    h2p, s2 = _tc_mid(r1, s1, b1_2, dinv16, W2, 128, 128)
    r2 = _make_agg(128)(h2p, edge)
    h3p, s3 = _tc_mid(r2, s2, b2_2, dinv16, W3, 128, 32)
    r3 = _make_agg(32)(h3p, edge)
    return _tc_fin(r3, s3, b3_2, dinv16)
